# baseline (speedup 1.0000x reference)
import functools

import jax
import jax.numpy as jnp
from jax import lax
from jax.experimental import pallas as pl
from jax.experimental.pallas import tpu as pltpu
from jax.experimental.pallas import tpu_sc as plsc

N_NODES = 10000
N_EDGES = 320000
D_FEAT = 128
N_ANS = 1000

NC = 2
NS = 16
NW = NC * NS
B = 128
CH = 16
NCH = 5
NB = CH * NCH
EPW = NB * B
TOT = NW * EPW
BT = TOT // B
NBT = BT // NS
CH2 = 32
NCH2 = NBT // CH2
R = N_NODES + 112
RPT = R // NS
RQ = R // B

def _deg_body(dst_hbm, ones_hbm, zeros_hbm, out0, dstv, onesv, dacc, sem):
    cid = lax.axis_index("c")
    sid = lax.axis_index("s")

    @pl.when(cid == 0)
    def _():
        pltpu.sync_copy(dst_hbm.at[pl.ds(sid * NBT, NBT)], dstv)
        pltpu.sync_copy(ones_hbm, onesv)

        @pl.when(sid == 0)
        def _():
            pltpu.sync_copy(zeros_hbm, dacc)

    plsc.subcore_barrier()

    @pl.when(cid == 0)
    def _():
        def fire(j, carry):
            pltpu.async_copy(onesv, dacc.at[dstv.at[j]], sem, add=True)
            return carry

        lax.fori_loop(0, NBT, fire, 0)

        def drain(j, carry):
            pltpu.make_async_copy(onesv, dacc.at[dstv.at[j]], sem).wait()
            return carry

        lax.fori_loop(0, NBT, drain, 0)

    plsc.subcore_barrier()

    @pl.when(jnp.logical_and(sid == 0, cid == 0))
    def _():
        pltpu.sync_copy(dacc, out0)


def _agg_body(src_hbm, dst_hbm, xs_hbm, out_hbm,
              srcv, dstv, bufa, bufb, acc, sema, semb):
    cid = lax.axis_index("c")
    sid = lax.axis_index("s")
    row0 = sid * RPT

    @pl.when(cid == 0)
    def _():
        def zrow(r, carry):
            for g in range(D_FEAT // 16):
                bufa[r, pl.ds(g * 16, 16)] = jnp.zeros((16,), jnp.float32)
            return carry

        lax.fori_loop(0, B, zrow, 0)
        for k in range(RPT // B):
            pltpu.sync_copy(bufa, acc.at[pl.ds(row0 + k * B, B)])
        rem = RPT - (RPT // B) * B
        pltpu.sync_copy(bufa.at[pl.ds(0, rem)],
                        acc.at[pl.ds(row0 + RPT - rem, rem)])

    plsc.subcore_barrier()

    @pl.when(cid == 0)
    def _():
        def chunk(c, carry):
            off = pl.multiple_of(sid * NBT + c * CH2, 8)
            pltpu.sync_copy(src_hbm.at[pl.ds(off, CH2)], srcv)
            pltpu.sync_copy(dst_hbm.at[pl.ds(off, CH2)], dstv)
            pltpu.async_copy(xs_hbm.at[srcv.at[0]], bufa, sema)

            def body(i, carry2):
                j = i * 2
                pltpu.async_copy(xs_hbm.at[srcv.at[j + 1]], bufb, semb)
                pltpu.make_async_copy(xs_hbm.at[srcv.at[j]], bufa,
                                      sema).wait()
                pltpu.sync_copy(bufa, acc.at[dstv.at[j]], add=True)
                pltpu.async_copy(xs_hbm.at[srcv.at[j + 2]], bufa, sema)
                pltpu.make_async_copy(xs_hbm.at[srcv.at[j + 1]], bufb,
                                      semb).wait()
                pltpu.sync_copy(bufb, acc.at[dstv.at[j + 1]], add=True)
                return carry2

            lax.fori_loop(0, CH2 // 2 - 1, body, 0)
            pltpu.async_copy(xs_hbm.at[srcv.at[CH2 - 1]], bufb, semb)
            pltpu.make_async_copy(xs_hbm.at[srcv.at[CH2 - 2]], bufa,
                                  sema).wait()
            pltpu.sync_copy(bufa, acc.at[dstv.at[CH2 - 2]], add=True)
            pltpu.make_async_copy(xs_hbm.at[srcv.at[CH2 - 1]], bufb,
                                  semb).wait()
            pltpu.sync_copy(bufb, acc.at[dstv.at[CH2 - 1]], add=True)
            return carry

        lax.fori_loop(0, NCH2, chunk, 0)

    plsc.subcore_barrier()

    @pl.when(cid == 0)
    def _():
        pltpu.sync_copy(acc.at[pl.ds(row0, RPT)],
                        out_hbm.at[pl.ds(row0, RPT)])


@functools.lru_cache(maxsize=None)
def _sc_kernels():
    mesh = plsc.VectorSubcoreMesh(core_axis_name="c", subcore_axis_name="s",
                                  num_cores=NC, num_subcores=NS)
    deg_k = pl.kernel(
        _deg_body,
        out_type=jax.ShapeDtypeStruct((R,), jnp.float32),
        mesh=mesh,
        scratch_types=[
            pltpu.VMEM((NBT, B), jnp.int32),
            pltpu.VMEM((B,), jnp.float32),
            pltpu.VMEM_SHARED((R,), jnp.float32),
            pltpu.SemaphoreType.DMA,
        ],
    )
    agg_k = pl.kernel(
        _agg_body,
        out_type=jax.ShapeDtypeStruct((R, D_FEAT), jnp.float32),
        mesh=mesh,
        scratch_types=[
            pltpu.VMEM((CH2, B), jnp.int32),
            pltpu.VMEM((CH2, B), jnp.int32),
            pltpu.VMEM((B, D_FEAT), jnp.float32),
            pltpu.VMEM((B, D_FEAT), jnp.float32),
            pltpu.VMEM_SHARED((R, D_FEAT), jnp.float32),
            pltpu.SemaphoreType.DMA,
            pltpu.SemaphoreType.DMA,
        ],
    )
    return deg_k, agg_k


def _scale_body(x_ref, d0_ref, o_ref):
    inv = lax.rsqrt(jnp.maximum(d0_ref[...], 1.0))
    o_ref[...] = x_ref[...] * inv


def _scale_x(x, d0):
    nblk = 10
    rows = N_NODES // nblk
    return pl.pallas_call(
        _scale_body,
        grid=(nblk,),
        in_specs=[
            pl.BlockSpec((rows, D_FEAT), lambda j: (j, 0)),
            pl.BlockSpec((rows, 1), lambda j: (j, 0)),
        ],
        out_specs=pl.BlockSpec((rows, D_FEAT), lambda j: (j, 0)),
        out_shape=jax.ShapeDtypeStruct((N_NODES, D_FEAT), jnp.float32),
    )(x, d0)


def _final_body(p0_ref, d0_ref, w_ref, bg_ref,
                w1_ref, b1_ref, w2_ref, b2_ref, o_ref, acc_ref, *, nblk):
    j = pl.program_id(0)
    inv = lax.rsqrt(jnp.maximum(d0_ref[...], 1.0))
    agg = p0_ref[...] * inv
    h = jnp.maximum(jnp.dot(agg, w_ref[...],
                            preferred_element_type=jnp.float32)
                    + bg_ref[...], 0.0)
    s = jnp.sum(h, axis=0, keepdims=True)

    @pl.when(j == 0)
    def _():
        acc_ref[0:1, :] = s

    @pl.when(j > 0)
    def _():
        acc_ref[0:1, :] = acc_ref[0:1, :] + s

    @pl.when(j == nblk - 1)
    def _():
        pooled = acc_ref[0:1, :] * (1.0 / N_NODES)
        z = jnp.dot(pooled, w1_ref[...],
                    preferred_element_type=jnp.float32) + b1_ref[...]
        z = jnp.dot(z, w2_ref[...],
                    preferred_element_type=jnp.float32) + b2_ref[...]
        m = jnp.max(z)
        lse = m + jnp.log(jnp.sum(jnp.exp(z - m)))
        o_ref[...] = z - lse


def _final(p0, d0, W_gcn, b_gcn, W_fuse1, b_fuse1, W_fuse2, b_fuse2):
    nblk = 10
    rows = N_NODES // nblk
    return pl.pallas_call(
        functools.partial(_final_body, nblk=nblk),
        grid=(nblk,),
        in_specs=[
            pl.BlockSpec((rows, D_FEAT), lambda j: (j, 0)),
            pl.BlockSpec((rows, 1), lambda j: (j, 0)),
            pl.BlockSpec((D_FEAT, D_FEAT), lambda j: (0, 0)),
            pl.BlockSpec((1, D_FEAT), lambda j: (0, 0)),
            pl.BlockSpec((D_FEAT, N_ANS), lambda j: (0, 0)),
            pl.BlockSpec((1, N_ANS), lambda j: (0, 0)),
            pl.BlockSpec((N_ANS, N_ANS), lambda j: (0, 0)),
            pl.BlockSpec((1, N_ANS), lambda j: (0, 0)),
        ],
        out_specs=pl.BlockSpec((1, N_ANS), lambda j: (0, 0)),
        out_shape=jax.ShapeDtypeStruct((1, N_ANS), jnp.float32),
        scratch_shapes=[pltpu.VMEM((8, D_FEAT), jnp.float32)],
    )(p0, d0, W_gcn, b_gcn, W_fuse1, b_fuse1, W_fuse2, b_fuse2)


def kernel(x, edge_index, W_gcn, b_gcn, W_fuse1, b_fuse1, W_fuse2, b_fuse2):
    src = edge_index[0].astype(jnp.int32)
    dst = edge_index[1].astype(jnp.int32)
    pad = TOT - N_EDGES
    srcp = jnp.concatenate([src, jnp.zeros((pad,), jnp.int32)])
    dstp = jnp.concatenate([dst, jnp.full((pad,), N_NODES, jnp.int32)])
    src_b = srcp.reshape(BT, B)
    dst_b = dstp.reshape(BT, B)

    ones_deg = jnp.ones((B,), jnp.float32)
    zeros_deg = jnp.zeros((R,), jnp.float32)

    deg_kernel, agg_kernel = _sc_kernels()
    d0g = deg_kernel(dst_b, ones_deg, zeros_deg)
    d0 = d0g.reshape(R, 1)

    xs = _scale_x(x, d0)

    aggp = agg_kernel(src_b, dst_b, xs)

    return _final(aggp, d0,
                  W_gcn, b_gcn.reshape(1, D_FEAT),
                  W_fuse1, b_fuse1.reshape(1, N_ANS),
                  W_fuse2, b_fuse2.reshape(1, N_ANS))


# speedup vs baseline: 1.0004x; 1.0004x over previous
import functools

import jax
import jax.numpy as jnp
from jax import lax
from jax.experimental import pallas as pl
from jax.experimental.pallas import tpu as pltpu
from jax.experimental.pallas import tpu_sc as plsc

N_NODES = 10000
N_EDGES = 320000
D_FEAT = 128
N_ANS = 1000

NC = 2
NS = 16
NW = NC * NS
B = 128
CH = 16
NCH = 5
NB = CH * NCH
EPW = NB * B
TOT = NW * EPW
BT = TOT // B
NBT = BT // NS
CH2 = 32
NCH2 = NBT // CH2
R = N_NODES + 112
RPT = R // NS
RQ = R // B

def _deg_body(dst_hbm, ones_hbm, zeros_hbm, out0, dstv, onesv, dacc, sem):
    cid = lax.axis_index("c")
    sid = lax.axis_index("s")

    @pl.when(cid == 0)
    def _():
        pltpu.sync_copy(dst_hbm.at[pl.ds(sid * NBT, NBT)], dstv)
        pltpu.sync_copy(ones_hbm, onesv)

        @pl.when(sid == 0)
        def _():
            pltpu.sync_copy(zeros_hbm, dacc)

    plsc.subcore_barrier()

    @pl.when(cid == 0)
    def _():
        def fire(j, carry):
            pltpu.async_copy(onesv, dacc.at[dstv.at[j]], sem, add=True)
            return carry

        lax.fori_loop(0, NBT, fire, 0)

        def drain(j, carry):
            pltpu.make_async_copy(onesv, dacc.at[dstv.at[j]], sem).wait()
            return carry

        lax.fori_loop(0, NBT, drain, 0)

    plsc.subcore_barrier()

    @pl.when(jnp.logical_and(sid == 0, cid == 0))
    def _():
        pltpu.sync_copy(dacc, out0)


def _agg_body(src_hbm, dst_hbm, xs_hbm, out_hbm,
              srcv, dstv, bufa, bufb, acc, sema, semb):
    cid = lax.axis_index("c")
    sid = lax.axis_index("s")
    row0 = sid * RPT

    def zrow(r, carry):
        for g in range(D_FEAT // 16):
            bufa[r, pl.ds(g * 16, 16)] = jnp.zeros((16,), jnp.float32)
        return carry

    lax.fori_loop(0, B, zrow, 0)
    for k in range(RPT // B):
        pltpu.sync_copy(bufa, acc.at[pl.ds(row0 + k * B, B)])
    rem = RPT - (RPT // B) * B
    pltpu.sync_copy(bufa.at[pl.ds(0, rem)],
                    acc.at[pl.ds(row0 + RPT - rem, rem)])

    plsc.subcore_barrier()

    nchunks = jnp.where(cid == 0, NCH2, 0)

    def chunk(c, carry):
        off = pl.multiple_of(sid * NBT + c * CH2, 8)
        pltpu.sync_copy(src_hbm.at[pl.ds(off, CH2)], srcv)
        pltpu.sync_copy(dst_hbm.at[pl.ds(off, CH2)], dstv)
        pltpu.async_copy(xs_hbm.at[srcv.at[0]], bufa, sema)

        def body(i, carry2):
            j = i * 2
            pltpu.async_copy(xs_hbm.at[srcv.at[j + 1]], bufb, semb)
            pltpu.make_async_copy(xs_hbm.at[srcv.at[j]], bufa,
                                  sema).wait()
            pltpu.sync_copy(bufa, acc.at[dstv.at[j]], add=True)
            pltpu.async_copy(xs_hbm.at[srcv.at[j + 2]], bufa, sema)
            pltpu.make_async_copy(xs_hbm.at[srcv.at[j + 1]], bufb,
                                  semb).wait()
            pltpu.sync_copy(bufb, acc.at[dstv.at[j + 1]], add=True)
            return carry2

        lax.fori_loop(0, CH2 // 2 - 1, body, 0)
        pltpu.async_copy(xs_hbm.at[srcv.at[CH2 - 1]], bufb, semb)
        pltpu.make_async_copy(xs_hbm.at[srcv.at[CH2 - 2]], bufa,
                              sema).wait()
        pltpu.sync_copy(bufa, acc.at[dstv.at[CH2 - 2]], add=True)
        pltpu.make_async_copy(xs_hbm.at[srcv.at[CH2 - 1]], bufb,
                              semb).wait()
        pltpu.sync_copy(bufb, acc.at[dstv.at[CH2 - 1]], add=True)
        return carry

    lax.fori_loop(0, nchunks, chunk, 0)

    plsc.subcore_barrier()

    @pl.when(cid == 0)
    def _():
        pltpu.sync_copy(acc.at[pl.ds(row0, RPT)],
                        out_hbm.at[pl.ds(row0, RPT)])


@functools.lru_cache(maxsize=None)
def _sc_kernels():
    mesh = plsc.VectorSubcoreMesh(core_axis_name="c", subcore_axis_name="s",
                                  num_cores=NC, num_subcores=NS)
    deg_k = pl.kernel(
        _deg_body,
        out_type=jax.ShapeDtypeStruct((R,), jnp.float32),
        mesh=mesh,
        scratch_types=[
            pltpu.VMEM((NBT, B), jnp.int32),
            pltpu.VMEM((B,), jnp.float32),
            pltpu.VMEM_SHARED((R,), jnp.float32),
            pltpu.SemaphoreType.DMA,
        ],
    )
    agg_k = pl.kernel(
        _agg_body,
        out_type=jax.ShapeDtypeStruct((R, D_FEAT), jnp.float32),
        mesh=mesh,
        scratch_types=[
            pltpu.VMEM((CH2, B), jnp.int32),
            pltpu.VMEM((CH2, B), jnp.int32),
            pltpu.VMEM((B, D_FEAT), jnp.float32),
            pltpu.VMEM((B, D_FEAT), jnp.float32),
            pltpu.VMEM_SHARED((R, D_FEAT), jnp.float32),
            pltpu.SemaphoreType.DMA,
            pltpu.SemaphoreType.DMA,
        ],
    )
    return deg_k, agg_k


def _scale_body(x_ref, d0_ref, o_ref):
    inv = lax.rsqrt(jnp.maximum(d0_ref[...], 1.0))
    o_ref[...] = x_ref[...] * inv


def _scale_x(x, d0):
    nblk = 10
    rows = N_NODES // nblk
    return pl.pallas_call(
        _scale_body,
        grid=(nblk,),
        in_specs=[
            pl.BlockSpec((rows, D_FEAT), lambda j: (j, 0)),
            pl.BlockSpec((rows, 1), lambda j: (j, 0)),
        ],
        out_specs=pl.BlockSpec((rows, D_FEAT), lambda j: (j, 0)),
        out_shape=jax.ShapeDtypeStruct((N_NODES, D_FEAT), jnp.float32),
    )(x, d0)


def _final_body(p0_ref, d0_ref, w_ref, bg_ref,
                w1_ref, b1_ref, w2_ref, b2_ref, o_ref, acc_ref, *, nblk):
    j = pl.program_id(0)
    inv = lax.rsqrt(jnp.maximum(d0_ref[...], 1.0))
    agg = p0_ref[...] * inv
    h = jnp.maximum(jnp.dot(agg, w_ref[...],
                            preferred_element_type=jnp.float32)
                    + bg_ref[...], 0.0)
    s = jnp.sum(h, axis=0, keepdims=True)

    @pl.when(j == 0)
    def _():
        acc_ref[0:1, :] = s

    @pl.when(j > 0)
    def _():
        acc_ref[0:1, :] = acc_ref[0:1, :] + s

    @pl.when(j == nblk - 1)
    def _():
        pooled = acc_ref[0:1, :] * (1.0 / N_NODES)
        z = jnp.dot(pooled, w1_ref[...],
                    preferred_element_type=jnp.float32) + b1_ref[...]
        z = jnp.dot(z, w2_ref[...],
                    preferred_element_type=jnp.float32) + b2_ref[...]
        m = jnp.max(z)
        lse = m + jnp.log(jnp.sum(jnp.exp(z - m)))
        o_ref[...] = z - lse


def _final(p0, d0, W_gcn, b_gcn, W_fuse1, b_fuse1, W_fuse2, b_fuse2):
    nblk = 10
    rows = N_NODES // nblk
    return pl.pallas_call(
        functools.partial(_final_body, nblk=nblk),
        grid=(nblk,),
        in_specs=[
            pl.BlockSpec((rows, D_FEAT), lambda j: (j, 0)),
            pl.BlockSpec((rows, 1), lambda j: (j, 0)),
            pl.BlockSpec((D_FEAT, D_FEAT), lambda j: (0, 0)),
            pl.BlockSpec((1, D_FEAT), lambda j: (0, 0)),
            pl.BlockSpec((D_FEAT, N_ANS), lambda j: (0, 0)),
            pl.BlockSpec((1, N_ANS), lambda j: (0, 0)),
            pl.BlockSpec((N_ANS, N_ANS), lambda j: (0, 0)),
            pl.BlockSpec((1, N_ANS), lambda j: (0, 0)),
        ],
        out_specs=pl.BlockSpec((1, N_ANS), lambda j: (0, 0)),
        out_shape=jax.ShapeDtypeStruct((1, N_ANS), jnp.float32),
        scratch_shapes=[pltpu.VMEM((8, D_FEAT), jnp.float32)],
    )(p0, d0, W_gcn, b_gcn, W_fuse1, b_fuse1, W_fuse2, b_fuse2)


def kernel(x, edge_index, W_gcn, b_gcn, W_fuse1, b_fuse1, W_fuse2, b_fuse2):
    src = edge_index[0].astype(jnp.int32)
    dst = edge_index[1].astype(jnp.int32)
    pad = TOT - N_EDGES
    srcp = jnp.concatenate([src, jnp.zeros((pad,), jnp.int32)])
    dstp = jnp.concatenate([dst, jnp.full((pad,), N_NODES, jnp.int32)])
    src_b = srcp.reshape(BT, B)
    dst_b = dstp.reshape(BT, B)

    ones_deg = jnp.ones((B,), jnp.float32)
    zeros_deg = jnp.zeros((R,), jnp.float32)

    deg_kernel, agg_kernel = _sc_kernels()
    d0g = deg_kernel(dst_b, ones_deg, zeros_deg)
    d0 = d0g.reshape(R, 1)

    xs = _scale_x(x, d0)

    aggp = agg_kernel(src_b, dst_b, xs)

    return _final(aggp, d0,
                  W_gcn, b_gcn.reshape(1, D_FEAT),
                  W_fuse1, b_fuse1.reshape(1, N_ANS),
                  W_fuse2, b_fuse2.reshape(1, N_ANS))


# speedup vs baseline: 1.0139x; 1.0135x over previous
import functools

import jax
import jax.numpy as jnp
from jax import lax
from jax.experimental import pallas as pl
from jax.experimental.pallas import tpu as pltpu
from jax.experimental.pallas import tpu_sc as plsc

N_NODES = 10000
N_EDGES = 320000
D_FEAT = 128
N_ANS = 1000

NC = 2
NS = 16
NW = NC * NS
B = 128
CH = 16
NCH = 5
NB = CH * NCH
EPW = NB * B
TOT = NW * EPW
BT = TOT // B
NBT = BT // NS
CH2 = 32
NCH2 = NBT // CH2
R = N_NODES + 112
RPT = R // NS
RQ = R // B

def _deg_body(dst_hbm, ones_hbm, zeros_hbm, out0, dstv, onesv, dacc, sem):
    cid = lax.axis_index("c")
    sid = lax.axis_index("s")

    @pl.when(cid == 0)
    def _():
        pltpu.sync_copy(dst_hbm.at[pl.ds(sid * NBT, NBT)], dstv)
        pltpu.sync_copy(ones_hbm, onesv)

        @pl.when(sid == 0)
        def _():
            pltpu.sync_copy(zeros_hbm, dacc)

    plsc.subcore_barrier()

    @pl.when(cid == 0)
    def _():
        def fire(j, carry):
            pltpu.async_copy(onesv, dacc.at[dstv.at[j]], sem, add=True)
            return carry

        lax.fori_loop(0, NBT, fire, 0)

        def drain(j, carry):
            pltpu.make_async_copy(onesv, dacc.at[dstv.at[j]], sem).wait()
            return carry

        lax.fori_loop(0, NBT, drain, 0)

    plsc.subcore_barrier()

    @pl.when(jnp.logical_and(sid == 0, cid == 0))
    def _():
        pltpu.sync_copy(dacc, out0)


def _agg_body(src_hbm, dst_hbm, xs_hbm, out_hbm,
              srcv, dstv, bufa, bufb, acc, sema, semb):
    cid = lax.axis_index("c")
    sid = lax.axis_index("s")
    row0 = sid * RPT

    def zrow(r, carry):
        for g in range(D_FEAT // 16):
            bufa[r, pl.ds(g * 16, 16)] = jnp.zeros((16,), jnp.float32)
        return carry

    lax.fori_loop(0, B, zrow, 0)
    for k in range(RPT // B):
        pltpu.sync_copy(bufa, acc.at[pl.ds(row0 + k * B, B)])
    rem = RPT - (RPT // B) * B
    pltpu.sync_copy(bufa.at[pl.ds(0, rem)],
                    acc.at[pl.ds(row0 + RPT - rem, rem)])

    plsc.subcore_barrier()

    nchunks = jnp.where(cid == 0, NCH2, 0)

    def chunk(c, carry):
        off = pl.multiple_of(sid * NBT + c * CH2, 8)
        pltpu.sync_copy(src_hbm.at[pl.ds(off, CH2)], srcv)
        pltpu.sync_copy(dst_hbm.at[pl.ds(off, CH2)], dstv)
        pltpu.async_copy(xs_hbm.at[srcv.at[0]], bufa, sema)

        def body(i, carry2):
            j = i * 2
            pltpu.async_copy(xs_hbm.at[srcv.at[j + 1]], bufb, semb)
            pltpu.make_async_copy(xs_hbm.at[srcv.at[j]], bufa,
                                  sema).wait()
            pltpu.sync_copy(bufa, acc.at[dstv.at[j]], add=True)
            pltpu.async_copy(xs_hbm.at[srcv.at[j + 2]], bufa, sema)
            pltpu.make_async_copy(xs_hbm.at[srcv.at[j + 1]], bufb,
                                  semb).wait()
            pltpu.sync_copy(bufb, acc.at[dstv.at[j + 1]], add=True)
            return carry2

        lax.fori_loop(0, CH2 // 2 - 1, body, 0)
        pltpu.async_copy(xs_hbm.at[srcv.at[CH2 - 1]], bufb, semb)
        pltpu.make_async_copy(xs_hbm.at[srcv.at[CH2 - 2]], bufa,
                              sema).wait()
        pltpu.sync_copy(bufa, acc.at[dstv.at[CH2 - 2]], add=True)
        pltpu.make_async_copy(xs_hbm.at[srcv.at[CH2 - 1]], bufb,
                              semb).wait()
        pltpu.sync_copy(bufb, acc.at[dstv.at[CH2 - 1]], add=True)
        return carry

    lax.fori_loop(0, nchunks, chunk, 0)

    plsc.subcore_barrier()

    @pl.when(cid == 0)
    def _():
        pltpu.sync_copy(acc.at[pl.ds(row0, RPT)],
                        out_hbm.at[pl.ds(row0, RPT)])


@functools.lru_cache(maxsize=None)
def _sc_kernels():
    mesh = plsc.VectorSubcoreMesh(core_axis_name="c", subcore_axis_name="s",
                                  num_cores=NC, num_subcores=NS)
    deg_k = pl.kernel(
        _deg_body,
        out_type=jax.ShapeDtypeStruct((R,), jnp.float32),
        mesh=mesh,
        scratch_types=[
            pltpu.VMEM((NBT, B), jnp.int32),
            pltpu.VMEM((B,), jnp.float32),
            pltpu.VMEM_SHARED((R,), jnp.float32),
            pltpu.SemaphoreType.DMA,
        ],
    )
    agg_k = pl.kernel(
        _agg_body,
        out_type=jax.ShapeDtypeStruct((R, D_FEAT), jnp.float32),
        mesh=mesh,
        scratch_types=[
            pltpu.VMEM((CH2, B), jnp.int32),
            pltpu.VMEM((CH2, B), jnp.int32),
            pltpu.VMEM((B, D_FEAT), jnp.float32),
            pltpu.VMEM((B, D_FEAT), jnp.float32),
            pltpu.VMEM_SHARED((R, D_FEAT), jnp.float32),
            pltpu.SemaphoreType.DMA,
            pltpu.SemaphoreType.DMA,
        ],
    )
    return deg_k, agg_k


def _scale_body(x_ref, d0_ref, o_ref):
    inv = lax.rsqrt(jnp.maximum(d0_ref[...], 1.0))
    o_ref[...] = x_ref[...] * inv


def _scale_x(x, d0):
    nblk = 10
    rows = N_NODES // nblk
    return pl.pallas_call(
        _scale_body,
        grid=(nblk,),
        in_specs=[
            pl.BlockSpec((rows, D_FEAT), lambda j: (j, 0)),
            pl.BlockSpec((rows, 1), lambda j: (j, 0)),
        ],
        out_specs=pl.BlockSpec((rows, D_FEAT), lambda j: (j, 0)),
        out_shape=jax.ShapeDtypeStruct((N_NODES, D_FEAT), jnp.float32),
    )(x, d0)


def _final_body(p0_ref, d0_ref, w_ref, bg_ref,
                w1_ref, b1_ref, w2_ref, b2_ref, o_ref, acc_ref, *, nblk):
    j = pl.program_id(0)
    inv = lax.rsqrt(jnp.maximum(d0_ref[...], 1.0))
    agg = p0_ref[...] * inv
    h = jnp.maximum(jnp.dot(agg, w_ref[...],
                            preferred_element_type=jnp.float32)
                    + bg_ref[...], 0.0)
    s = jnp.sum(h, axis=0, keepdims=True)

    @pl.when(j == 0)
    def _():
        acc_ref[0:1, :] = s

    @pl.when(j > 0)
    def _():
        acc_ref[0:1, :] = acc_ref[0:1, :] + s

    @pl.when(j == nblk - 1)
    def _():
        pooled = acc_ref[0:1, :] * (1.0 / N_NODES)
        z = jnp.dot(pooled, w1_ref[...],
                    preferred_element_type=jnp.float32) + b1_ref[...]
        z = jnp.dot(z, w2_ref[...],
                    preferred_element_type=jnp.float32) + b2_ref[...]
        m = jnp.max(z)
        lse = m + jnp.log(jnp.sum(jnp.exp(z - m)))
        o_ref[...] = z - lse


def _final(p0, d0, W_gcn, b_gcn, W_fuse1, b_fuse1, W_fuse2, b_fuse2):
    nblk = 10
    rows = N_NODES // nblk
    return pl.pallas_call(
        functools.partial(_final_body, nblk=nblk),
        grid=(nblk,),
        in_specs=[
            pl.BlockSpec((rows, D_FEAT), lambda j: (j, 0)),
            pl.BlockSpec((rows, 1), lambda j: (j, 0)),
            pl.BlockSpec((D_FEAT, D_FEAT), lambda j: (0, 0)),
            pl.BlockSpec((1, D_FEAT), lambda j: (0, 0)),
            pl.BlockSpec((D_FEAT, N_ANS), lambda j: (0, 0)),
            pl.BlockSpec((1, N_ANS), lambda j: (0, 0)),
            pl.BlockSpec((N_ANS, N_ANS), lambda j: (0, 0)),
            pl.BlockSpec((1, N_ANS), lambda j: (0, 0)),
        ],
        out_specs=pl.BlockSpec((1, N_ANS), lambda j: (0, 0)),
        out_shape=jax.ShapeDtypeStruct((1, N_ANS), jnp.float32),
        scratch_shapes=[pltpu.VMEM((8, D_FEAT), jnp.float32)],
    )(p0, d0, W_gcn, b_gcn, W_fuse1, b_fuse1, W_fuse2, b_fuse2)


def kernel(x, edge_index, W_gcn, b_gcn, W_fuse1, b_fuse1, W_fuse2, b_fuse2):
    src = edge_index[0].astype(jnp.int32)
    dst = edge_index[1].astype(jnp.int32)
    pad = TOT - N_EDGES
    srcp = jnp.concatenate([src, jnp.zeros((pad,), jnp.int32)])
    trash = N_NODES + jnp.arange(pad, dtype=jnp.int32) % (R - N_NODES)
    dstp = jnp.concatenate([dst, trash])
    src_b = srcp.reshape(BT, B)
    dst_b = dstp.reshape(BT, B)

    ones_deg = jnp.ones((B,), jnp.float32)
    zeros_deg = jnp.zeros((R,), jnp.float32)

    deg_kernel, agg_kernel = _sc_kernels()
    d0g = deg_kernel(dst_b, ones_deg, zeros_deg)
    d0 = d0g.reshape(R, 1)

    xs = _scale_x(x, d0)

    aggp = agg_kernel(src_b, dst_b, xs)

    return _final(aggp, d0,
                  W_gcn, b_gcn.reshape(1, D_FEAT),
                  W_fuse1, b_fuse1.reshape(1, N_ANS),
                  W_fuse2, b_fuse2.reshape(1, N_ANS))


# speedup vs baseline: 2.1528x; 2.1232x over previous
import functools

import jax
import jax.numpy as jnp
from jax import lax
from jax.experimental import pallas as pl
from jax.experimental.pallas import tpu as pltpu
from jax.experimental.pallas import tpu_sc as plsc

N_NODES = 10000
N_EDGES = 320000
D_FEAT = 128
N_ANS = 1000

NC = 2
NS = 16
NW = NC * NS
B = 128
CH = 16
NCH = 5
NB = CH * NCH
EPW = NB * B
TOT = NW * EPW
BT = TOT // B
NBT = BT // NS
CH2 = 32
NCH2 = NBT // CH2
R = N_NODES + 112
RPT = R // NS
RQ = R // B

def _deg_body(dst_hbm, ones_hbm, zeros_hbm, out0, dstv, onesv, dacc, sem):
    cid = lax.axis_index("c")
    sid = lax.axis_index("s")

    @pl.when(cid == 0)
    def _():
        pltpu.sync_copy(dst_hbm.at[pl.ds(sid * NBT, NBT)], dstv)
        pltpu.sync_copy(ones_hbm, onesv)

        @pl.when(sid == 0)
        def _():
            pltpu.sync_copy(zeros_hbm, dacc)

    plsc.subcore_barrier()

    @pl.when(cid == 0)
    def _():
        def fire(j, carry):
            pltpu.async_copy(onesv, dacc.at[dstv.at[j]], sem, add=True)
            return carry

        lax.fori_loop(0, NBT, fire, 0)

        def drain(j, carry):
            pltpu.make_async_copy(onesv, dacc.at[dstv.at[j]], sem).wait()
            return carry

        lax.fori_loop(0, NBT, drain, 0)

    plsc.subcore_barrier()

    @pl.when(jnp.logical_and(sid == 0, cid == 0))
    def _():
        pltpu.sync_copy(dacc, out0)


def _agg_body(src_hbm, dst_hbm, xs_hbm, out_hbm,
              srcv, dstv, bufa, bufb, acc, sema, semb):
    cid = lax.axis_index("c")
    sid = lax.axis_index("s")
    row0 = sid * RPT

    def zrow(r, carry):
        for g in range(D_FEAT // 16):
            bufa[r, pl.ds(g * 16, 16)] = jnp.zeros((16,), jnp.float32)
        return carry

    lax.fori_loop(0, B, zrow, 0)
    for k in range(RPT // B):
        pltpu.sync_copy(bufa, acc.at[pl.ds(row0 + k * B, B)])
    rem = RPT - (RPT // B) * B
    pltpu.sync_copy(bufa.at[pl.ds(0, rem)],
                    acc.at[pl.ds(row0 + RPT - rem, rem)])

    plsc.subcore_barrier()

    nchunks = jnp.where(cid == 0, NCH2, 0)

    def chunk(c, carry):
        off = pl.multiple_of(sid * NBT + c * CH2, 8)
        pltpu.sync_copy(src_hbm.at[pl.ds(off, CH2)], srcv)
        pltpu.sync_copy(dst_hbm.at[pl.ds(off, CH2)], dstv)
        pltpu.async_copy(xs_hbm.at[srcv.at[0]], bufa, sema)

        def body(i, carry2):
            j = i * 2
            pltpu.async_copy(xs_hbm.at[srcv.at[j + 1]], bufb, semb)
            pltpu.make_async_copy(xs_hbm.at[srcv.at[j]], bufa,
                                  sema).wait()
            pltpu.sync_copy(bufa, acc.at[dstv.at[j]], add=True)
            pltpu.async_copy(xs_hbm.at[srcv.at[j + 2]], bufa, sema)
            pltpu.make_async_copy(xs_hbm.at[srcv.at[j + 1]], bufb,
                                  semb).wait()
            pltpu.sync_copy(bufb, acc.at[dstv.at[j + 1]], add=True)
            return carry2

        lax.fori_loop(0, CH2 // 2 - 1, body, 0)
        pltpu.async_copy(xs_hbm.at[srcv.at[CH2 - 1]], bufb, semb)
        pltpu.make_async_copy(xs_hbm.at[srcv.at[CH2 - 2]], bufa,
                              sema).wait()
        pltpu.sync_copy(bufa, acc.at[dstv.at[CH2 - 2]], add=True)
        pltpu.make_async_copy(xs_hbm.at[srcv.at[CH2 - 1]], bufb,
                              semb).wait()
        pltpu.sync_copy(bufb, acc.at[dstv.at[CH2 - 1]], add=True)
        return carry

    lax.fori_loop(0, nchunks, chunk, 0)

    plsc.subcore_barrier()

    @pl.when(cid == 0)
    def _():
        pltpu.sync_copy(acc.at[pl.ds(row0, RPT)],
                        out_hbm.at[pl.ds(row0, RPT)])


@functools.lru_cache(maxsize=None)
def _sc_kernels():
    mesh = plsc.VectorSubcoreMesh(core_axis_name="c", subcore_axis_name="s",
                                  num_cores=NC, num_subcores=NS)
    deg_k = pl.kernel(
        _deg_body,
        out_type=jax.ShapeDtypeStruct((R,), jnp.float32),
        mesh=mesh,
        scratch_types=[
            pltpu.VMEM((NBT, B), jnp.int32),
            pltpu.VMEM((B,), jnp.float32),
            pltpu.VMEM_SHARED((R,), jnp.float32),
            pltpu.SemaphoreType.DMA,
        ],
    )
    agg_k = pl.kernel(
        _agg_body,
        out_type=jax.ShapeDtypeStruct((R, D_FEAT), jnp.float32),
        mesh=mesh,
        scratch_types=[
            pltpu.VMEM((CH2, B), jnp.int32),
            pltpu.VMEM((CH2, B), jnp.int32),
            pltpu.VMEM((B, D_FEAT), jnp.float32),
            pltpu.VMEM((B, D_FEAT), jnp.float32),
            pltpu.VMEM_SHARED((R, D_FEAT), jnp.float32),
            pltpu.SemaphoreType.DMA,
            pltpu.SemaphoreType.DMA,
        ],
    )
    return deg_k, agg_k


def _scale_body(x_ref, d0_ref, o_ref):
    inv = lax.rsqrt(jnp.maximum(d0_ref[...], 1.0))
    o_ref[...] = x_ref[...] * inv


def _scale_x(x, d0):
    nblk = 10
    rows = N_NODES // nblk
    return pl.pallas_call(
        _scale_body,
        grid=(nblk,),
        in_specs=[
            pl.BlockSpec((rows, D_FEAT), lambda j: (j, 0)),
            pl.BlockSpec((rows, 1), lambda j: (j, 0)),
        ],
        out_specs=pl.BlockSpec((rows, D_FEAT), lambda j: (j, 0)),
        out_shape=jax.ShapeDtypeStruct((N_NODES, D_FEAT), jnp.float32),
    )(x, d0)


def _final_body(p0_ref, d0_ref, w_ref, bg_ref,
                w1_ref, b1_ref, w2_ref, b2_ref, o_ref, acc_ref, *, nblk):
    j = pl.program_id(0)
    inv = lax.rsqrt(jnp.maximum(d0_ref[...], 1.0))
    agg = p0_ref[...] * inv
    h = jnp.maximum(jnp.dot(agg, w_ref[...],
                            preferred_element_type=jnp.float32)
                    + bg_ref[...], 0.0)
    s = jnp.sum(h, axis=0, keepdims=True)

    @pl.when(j == 0)
    def _():
        acc_ref[0:1, :] = s

    @pl.when(j > 0)
    def _():
        acc_ref[0:1, :] = acc_ref[0:1, :] + s

    @pl.when(j == nblk - 1)
    def _():
        pooled = acc_ref[0:1, :] * (1.0 / N_NODES)
        z = jnp.dot(pooled, w1_ref[...],
                    preferred_element_type=jnp.float32) + b1_ref[...]
        z = jnp.dot(z, w2_ref[...],
                    preferred_element_type=jnp.float32) + b2_ref[...]
        m = jnp.max(z)
        lse = m + jnp.log(jnp.sum(jnp.exp(z - m)))
        o_ref[...] = z - lse


def _final(p0, d0, W_gcn, b_gcn, W_fuse1, b_fuse1, W_fuse2, b_fuse2):
    nblk = 10
    rows = N_NODES // nblk
    return pl.pallas_call(
        functools.partial(_final_body, nblk=nblk),
        grid=(nblk,),
        in_specs=[
            pl.BlockSpec((rows, D_FEAT), lambda j: (j, 0)),
            pl.BlockSpec((rows, 1), lambda j: (j, 0)),
            pl.BlockSpec((D_FEAT, D_FEAT), lambda j: (0, 0)),
            pl.BlockSpec((1, D_FEAT), lambda j: (0, 0)),
            pl.BlockSpec((D_FEAT, N_ANS), lambda j: (0, 0)),
            pl.BlockSpec((1, N_ANS), lambda j: (0, 0)),
            pl.BlockSpec((N_ANS, N_ANS), lambda j: (0, 0)),
            pl.BlockSpec((1, N_ANS), lambda j: (0, 0)),
        ],
        out_specs=pl.BlockSpec((1, N_ANS), lambda j: (0, 0)),
        out_shape=jax.ShapeDtypeStruct((1, N_ANS), jnp.float32),
        scratch_shapes=[pltpu.VMEM((8, D_FEAT), jnp.float32)],
    )(p0, d0, W_gcn, b_gcn, W_fuse1, b_fuse1, W_fuse2, b_fuse2)


def kernel(x, edge_index, W_gcn, b_gcn, W_fuse1, b_fuse1, W_fuse2, b_fuse2):
    src = edge_index[0].astype(jnp.int32)
    dst = edge_index[1].astype(jnp.int32)
    pad = TOT - N_EDGES
    srcp = jnp.concatenate([src, jnp.arange(pad, dtype=jnp.int32) % N_NODES])
    trash = N_NODES + jnp.arange(pad, dtype=jnp.int32) % (R - N_NODES)
    dstp = jnp.concatenate([dst, trash])
    src_b = srcp.reshape(BT, B)
    dst_b = dstp.reshape(BT, B)

    ones_deg = jnp.ones((B,), jnp.float32)
    zeros_deg = jnp.zeros((R,), jnp.float32)

    deg_kernel, agg_kernel = _sc_kernels()
    d0g = deg_kernel(dst_b, ones_deg, zeros_deg)
    d0 = d0g.reshape(R, 1)

    xs = _scale_x(x, d0)

    aggp = agg_kernel(src_b, dst_b, xs)

    return _final(aggp, d0,
                  W_gcn, b_gcn.reshape(1, D_FEAT),
                  W_fuse1, b_fuse1.reshape(1, N_ANS),
                  W_fuse2, b_fuse2.reshape(1, N_ANS))


# speedup vs baseline: 3.0776x; 1.4296x over previous
import functools

import jax
import jax.numpy as jnp
from jax import lax
from jax.experimental import pallas as pl
from jax.experimental.pallas import tpu as pltpu
from jax.experimental.pallas import tpu_sc as plsc

N_NODES = 10000
N_EDGES = 320000
D_FEAT = 128
N_ANS = 1000

NC = 2
NS = 16
NW = NC * NS
B = 128
NBW = 80
BT = NW * NBW
TOT = BT * B
CH2 = 40
NCH2 = NBW // CH2
R = N_NODES + 112
RPT = R // NS


def _deg_body(dst_hbm, ones_hbm, zeros_hbm, out0, out1, dstv, onesv, dacc,
              sem):
    cid = lax.axis_index("c")
    sid = lax.axis_index("s")
    wid = cid * NS + sid
    pltpu.sync_copy(dst_hbm.at[pl.ds(wid * NBW, NBW)], dstv)
    pltpu.sync_copy(ones_hbm, onesv)

    @pl.when(sid == 0)
    def _():
        pltpu.sync_copy(zeros_hbm, dacc)

    plsc.subcore_barrier()

    def fire(j, carry):
        pltpu.async_copy(onesv, dacc.at[dstv.at[j]], sem, add=True)
        return carry

    lax.fori_loop(0, NBW, fire, 0)

    def drain(j, carry):
        pltpu.make_async_copy(onesv, dacc.at[dstv.at[j]], sem).wait()
        return carry

    lax.fori_loop(0, NBW, drain, 0)
    plsc.subcore_barrier()

    @pl.when(jnp.logical_and(sid == 0, cid == 0))
    def _():
        pltpu.sync_copy(dacc, out0)

    @pl.when(jnp.logical_and(sid == 0, cid == 1))
    def _():
        pltpu.sync_copy(dacc, out1)


def _agg_body(src_hbm, dst_hbm, xs_hbm, out_hbm,
              srcv, dstv, bufa, bufb, acc, sema, semb):
    cid = lax.axis_index("c")
    sid = lax.axis_index("s")
    wid = cid * NS + sid
    row0 = sid * RPT

    def zrow(r, carry):
        for g in range(D_FEAT // 16):
            bufa[r, pl.ds(g * 16, 16)] = jnp.zeros((16,), jnp.float32)
        return carry

    lax.fori_loop(0, B, zrow, 0)
    for k in range(RPT // B):
        pltpu.sync_copy(bufa, acc.at[pl.ds(row0 + k * B, B)])
    rem = RPT - (RPT // B) * B
    pltpu.sync_copy(bufa.at[pl.ds(0, rem)],
                    acc.at[pl.ds(row0 + RPT - rem, rem)])

    plsc.subcore_barrier()

    def chunk(c, carry):
        off = pl.multiple_of(wid * NBW + c * CH2, 8)
        pltpu.sync_copy(src_hbm.at[pl.ds(off, CH2)], srcv)
        pltpu.sync_copy(dst_hbm.at[pl.ds(off, CH2)], dstv)
        pltpu.async_copy(xs_hbm.at[srcv.at[0]], bufa, sema)

        def body(i, carry2):
            j = i * 2
            pltpu.async_copy(xs_hbm.at[srcv.at[j + 1]], bufb, semb)
            pltpu.make_async_copy(xs_hbm.at[srcv.at[j]], bufa, sema).wait()
            pltpu.sync_copy(bufa, acc.at[dstv.at[j]], add=True)
            pltpu.async_copy(xs_hbm.at[srcv.at[j + 2]], bufa, sema)
            pltpu.make_async_copy(xs_hbm.at[srcv.at[j + 1]], bufb,
                                  semb).wait()
            pltpu.sync_copy(bufb, acc.at[dstv.at[j + 1]], add=True)
            return carry2

        lax.fori_loop(0, CH2 // 2 - 1, body, 0)
        pltpu.async_copy(xs_hbm.at[srcv.at[CH2 - 1]], bufb, semb)
        pltpu.make_async_copy(xs_hbm.at[srcv.at[CH2 - 2]], bufa, sema).wait()
        pltpu.sync_copy(bufa, acc.at[dstv.at[CH2 - 2]], add=True)
        pltpu.make_async_copy(xs_hbm.at[srcv.at[CH2 - 1]], bufb, semb).wait()
        pltpu.sync_copy(bufb, acc.at[dstv.at[CH2 - 1]], add=True)
        return carry

    lax.fori_loop(0, NCH2, chunk, 0)
    plsc.subcore_barrier()
    pltpu.sync_copy(acc.at[pl.ds(row0, RPT)],
                    out_hbm.at[cid, pl.ds(row0, RPT)])


@functools.lru_cache(maxsize=None)
def _sc_kernels():
    mesh = plsc.VectorSubcoreMesh(core_axis_name="c", subcore_axis_name="s",
                                  num_cores=NC, num_subcores=NS)
    deg_k = pl.kernel(
        _deg_body,
        out_type=(jax.ShapeDtypeStruct((R,), jnp.float32),
                  jax.ShapeDtypeStruct((R,), jnp.float32)),
        mesh=mesh,
        scratch_types=[
            pltpu.VMEM((NBW, B), jnp.int32),
            pltpu.VMEM((B,), jnp.float32),
            pltpu.VMEM_SHARED((R,), jnp.float32),
            pltpu.SemaphoreType.DMA,
        ],
    )
    agg_k = pl.kernel(
        _agg_body,
        out_type=jax.ShapeDtypeStruct((NC, R, D_FEAT), jnp.float32),
        mesh=mesh,
        scratch_types=[
            pltpu.VMEM((CH2, B), jnp.int32),
            pltpu.VMEM((CH2, B), jnp.int32),
            pltpu.VMEM((B, D_FEAT), jnp.float32),
            pltpu.VMEM((B, D_FEAT), jnp.float32),
            pltpu.VMEM_SHARED((R, D_FEAT), jnp.float32),
            pltpu.SemaphoreType.DMA,
            pltpu.SemaphoreType.DMA,
        ],
    )
    return deg_k, agg_k


def _scale_body(x_ref, d0_ref, d1_ref, o_ref):
    deg = d0_ref[...] + d1_ref[...]
    inv = lax.rsqrt(jnp.maximum(deg, 1.0))
    o_ref[...] = x_ref[...] * inv


def _scale_x(x, d0, d1):
    nblk = 10
    rows = N_NODES // nblk
    return pl.pallas_call(
        _scale_body,
        grid=(nblk,),
        in_specs=[
            pl.BlockSpec((rows, D_FEAT), lambda j: (j, 0)),
            pl.BlockSpec((rows, 1), lambda j: (j, 0)),
            pl.BlockSpec((rows, 1), lambda j: (j, 0)),
        ],
        out_specs=pl.BlockSpec((rows, D_FEAT), lambda j: (j, 0)),
        out_shape=jax.ShapeDtypeStruct((N_NODES, D_FEAT), jnp.float32),
    )(x, d0, d1)


def _final_body(p0_ref, p1_ref, d0_ref, d1_ref, w_ref, bg_ref,
                w1_ref, b1_ref, w2_ref, b2_ref, o_ref, acc_ref, *, nblk):
    j = pl.program_id(0)
    deg = d0_ref[...] + d1_ref[...]
    inv = lax.rsqrt(jnp.maximum(deg, 1.0))
    agg = (p0_ref[...] + p1_ref[...]) * inv
    h = jnp.maximum(jnp.dot(agg, w_ref[...],
                            preferred_element_type=jnp.float32)
                    + bg_ref[...], 0.0)
    s = jnp.sum(h, axis=0, keepdims=True)

    @pl.when(j == 0)
    def _():
        acc_ref[0:1, :] = s

    @pl.when(j > 0)
    def _():
        acc_ref[0:1, :] = acc_ref[0:1, :] + s

    @pl.when(j == nblk - 1)
    def _():
        pooled = acc_ref[0:1, :] * (1.0 / N_NODES)
        z = jnp.dot(pooled, w1_ref[...],
                    preferred_element_type=jnp.float32) + b1_ref[...]
        z = jnp.dot(z, w2_ref[...],
                    preferred_element_type=jnp.float32) + b2_ref[...]
        m = jnp.max(z)
        lse = m + jnp.log(jnp.sum(jnp.exp(z - m)))
        o_ref[...] = z - lse


def _final(p0, p1, d0, d1, W_gcn, b_gcn, W_fuse1, b_fuse1, W_fuse2, b_fuse2):
    nblk = 10
    rows = N_NODES // nblk
    return pl.pallas_call(
        functools.partial(_final_body, nblk=nblk),
        grid=(nblk,),
        in_specs=[
            pl.BlockSpec((rows, D_FEAT), lambda j: (j, 0)),
            pl.BlockSpec((rows, D_FEAT), lambda j: (j, 0)),
            pl.BlockSpec((rows, 1), lambda j: (j, 0)),
            pl.BlockSpec((rows, 1), lambda j: (j, 0)),
            pl.BlockSpec((D_FEAT, D_FEAT), lambda j: (0, 0)),
            pl.BlockSpec((1, D_FEAT), lambda j: (0, 0)),
            pl.BlockSpec((D_FEAT, N_ANS), lambda j: (0, 0)),
            pl.BlockSpec((1, N_ANS), lambda j: (0, 0)),
            pl.BlockSpec((N_ANS, N_ANS), lambda j: (0, 0)),
            pl.BlockSpec((1, N_ANS), lambda j: (0, 0)),
        ],
        out_specs=pl.BlockSpec((1, N_ANS), lambda j: (0, 0)),
        out_shape=jax.ShapeDtypeStruct((1, N_ANS), jnp.float32),
        scratch_shapes=[pltpu.VMEM((8, D_FEAT), jnp.float32)],
    )(p0, p1, d0, d1, W_gcn, b_gcn, W_fuse1, b_fuse1, W_fuse2, b_fuse2)


def kernel(x, edge_index, W_gcn, b_gcn, W_fuse1, b_fuse1, W_fuse2, b_fuse2):
    src = edge_index[0].astype(jnp.int32)
    dst = edge_index[1].astype(jnp.int32)
    pad = TOT - N_EDGES
    srcp = jnp.concatenate([src, jnp.arange(pad, dtype=jnp.int32) % N_NODES])
    trash = N_NODES + jnp.arange(pad, dtype=jnp.int32) % (R - N_NODES)
    dstp = jnp.concatenate([dst, trash])
    src_b = srcp.reshape(BT, B)
    dst_b = dstp.reshape(BT, B)

    ones_deg = jnp.ones((B,), jnp.float32)
    zeros_deg = jnp.zeros((R,), jnp.float32)

    deg_kernel, agg_kernel = _sc_kernels()
    d0g, d1g = deg_kernel(dst_b, ones_deg, zeros_deg)
    d0 = d0g.reshape(R, 1)
    d1 = d1g.reshape(R, 1)

    xs = _scale_x(x, d0, d1)

    aggp = agg_kernel(src_b, dst_b, xs)

    return _final(aggp[0], aggp[1], d0, d1,
                  W_gcn, b_gcn.reshape(1, D_FEAT),
                  W_fuse1, b_fuse1.reshape(1, N_ANS),
                  W_fuse2, b_fuse2.reshape(1, N_ANS))


# speedup vs baseline: 3.1627x; 1.0276x over previous
import functools

import jax
import jax.numpy as jnp
from jax import lax
from jax.experimental import pallas as pl
from jax.experimental.pallas import tpu as pltpu
from jax.experimental.pallas import tpu_sc as plsc

N_NODES = 10000
N_EDGES = 320000
D_FEAT = 128
N_ANS = 1000

NC = 2
NS = 16
NW = NC * NS
B = 128
NBW = 80
BT = NW * NBW
TOT = BT * B
BA = 64
NBWA = NBW * (B // BA)
CHA = 40
NCHA = NBWA // CHA
R = N_NODES + 112
RPT = R // NS


def _deg_body(dst_hbm, ones_hbm, zeros_hbm, out0, out1, dstv, onesv, dacc,
              sem):
    cid = lax.axis_index("c")
    sid = lax.axis_index("s")
    wid = cid * NS + sid
    pltpu.sync_copy(dst_hbm.at[pl.ds(wid * NBW, NBW)], dstv)
    pltpu.sync_copy(ones_hbm, onesv)

    @pl.when(sid == 0)
    def _():
        pltpu.sync_copy(zeros_hbm, dacc)

    plsc.subcore_barrier()

    def fire(j, carry):
        pltpu.async_copy(onesv, dacc.at[dstv.at[j]], sem, add=True)
        return carry

    lax.fori_loop(0, NBW, fire, 0)

    def drain(j, carry):
        pltpu.make_async_copy(onesv, dacc.at[dstv.at[j]], sem).wait()
        return carry

    lax.fori_loop(0, NBW, drain, 0)
    plsc.subcore_barrier()

    @pl.when(jnp.logical_and(sid == 0, cid == 0))
    def _():
        pltpu.sync_copy(dacc, out0)

    @pl.when(jnp.logical_and(sid == 0, cid == 1))
    def _():
        pltpu.sync_copy(dacc, out1)


def _agg_body(src_hbm, dst_hbm, xs_hbm, out_hbm,
              srcv, dstv, buf0, buf1, buf2, buf3, acc,
              sem0, sem1, sem2, sem3):
    cid = lax.axis_index("c")
    sid = lax.axis_index("s")
    wid = cid * NS + sid
    row0 = sid * RPT
    bufs = (buf0, buf1, buf2, buf3)
    sems = (sem0, sem1, sem2, sem3)

    def zrow(r, carry):
        for g in range(D_FEAT // 16):
            buf0[r, pl.ds(g * 16, 16)] = jnp.zeros((16,), jnp.float32)
        return carry

    lax.fori_loop(0, BA, zrow, 0)
    for k in range(RPT // BA):
        pltpu.sync_copy(buf0, acc.at[pl.ds(row0 + k * BA, BA)])
    rem = RPT - (RPT // BA) * BA
    if rem:
        pltpu.sync_copy(buf0.at[pl.ds(0, rem)],
                        acc.at[pl.ds(row0 + RPT - rem, rem)])

    plsc.subcore_barrier()

    def chunk(c, carry):
        off = pl.multiple_of(wid * NBWA + c * CHA, 8)
        pltpu.sync_copy(src_hbm.at[pl.ds(off, CHA)], srcv)
        pltpu.sync_copy(dst_hbm.at[pl.ds(off, CHA)], dstv)
        for k in range(4):
            pltpu.async_copy(xs_hbm.at[srcv.at[k]], bufs[k], sems[k])

        def body(i, carry2):
            j = i * 4
            for k in range(4):
                pltpu.make_async_copy(xs_hbm.at[srcv.at[j + k]], bufs[k],
                                      sems[k]).wait()
                pltpu.sync_copy(bufs[k], acc.at[dstv.at[j + k]], add=True)
                pltpu.async_copy(xs_hbm.at[srcv.at[j + k + 4]], bufs[k],
                                 sems[k])
            return carry2

        lax.fori_loop(0, CHA // 4 - 1, body, 0)
        j0 = CHA - 4
        for k in range(4):
            pltpu.make_async_copy(xs_hbm.at[srcv.at[j0 + k]], bufs[k],
                                  sems[k]).wait()
            pltpu.sync_copy(bufs[k], acc.at[dstv.at[j0 + k]], add=True)
        return carry

    lax.fori_loop(0, NCHA, chunk, 0)
    plsc.subcore_barrier()
    pltpu.sync_copy(acc.at[pl.ds(row0, RPT)],
                    out_hbm.at[cid, pl.ds(row0, RPT)])


@functools.lru_cache(maxsize=None)
def _sc_kernels():
    mesh = plsc.VectorSubcoreMesh(core_axis_name="c", subcore_axis_name="s",
                                  num_cores=NC, num_subcores=NS)
    deg_k = pl.kernel(
        _deg_body,
        out_type=(jax.ShapeDtypeStruct((R,), jnp.float32),
                  jax.ShapeDtypeStruct((R,), jnp.float32)),
        mesh=mesh,
        scratch_types=[
            pltpu.VMEM((NBW, B), jnp.int32),
            pltpu.VMEM((B,), jnp.float32),
            pltpu.VMEM_SHARED((R,), jnp.float32),
            pltpu.SemaphoreType.DMA,
        ],
    )
    agg_k = pl.kernel(
        _agg_body,
        out_type=jax.ShapeDtypeStruct((NC, R, D_FEAT), jnp.float32),
        mesh=mesh,
        scratch_types=[
            pltpu.VMEM((CHA, BA), jnp.int32),
            pltpu.VMEM((CHA, BA), jnp.int32),
            pltpu.VMEM((BA, D_FEAT), jnp.float32),
            pltpu.VMEM((BA, D_FEAT), jnp.float32),
            pltpu.VMEM((BA, D_FEAT), jnp.float32),
            pltpu.VMEM((BA, D_FEAT), jnp.float32),
            pltpu.VMEM_SHARED((R, D_FEAT), jnp.float32),
            pltpu.SemaphoreType.DMA,
            pltpu.SemaphoreType.DMA,
            pltpu.SemaphoreType.DMA,
            pltpu.SemaphoreType.DMA,
        ],
    )
    return deg_k, agg_k


def _scale_body(x_ref, d0_ref, d1_ref, o_ref):
    deg = d0_ref[...] + d1_ref[...]
    inv = lax.rsqrt(jnp.maximum(deg, 1.0))
    o_ref[...] = x_ref[...] * inv


def _scale_x(x, d0, d1):
    nblk = 10
    rows = N_NODES // nblk
    return pl.pallas_call(
        _scale_body,
        grid=(nblk,),
        in_specs=[
            pl.BlockSpec((rows, D_FEAT), lambda j: (j, 0)),
            pl.BlockSpec((rows, 1), lambda j: (j, 0)),
            pl.BlockSpec((rows, 1), lambda j: (j, 0)),
        ],
        out_specs=pl.BlockSpec((rows, D_FEAT), lambda j: (j, 0)),
        out_shape=jax.ShapeDtypeStruct((N_NODES, D_FEAT), jnp.float32),
    )(x, d0, d1)


def _final_body(p0_ref, p1_ref, d0_ref, d1_ref, w_ref, bg_ref,
                w1_ref, b1_ref, w2_ref, b2_ref, o_ref, acc_ref, *, nblk):
    j = pl.program_id(0)
    deg = d0_ref[...] + d1_ref[...]
    inv = lax.rsqrt(jnp.maximum(deg, 1.0))
    agg = (p0_ref[...] + p1_ref[...]) * inv
    h = jnp.maximum(jnp.dot(agg, w_ref[...],
                            preferred_element_type=jnp.float32)
                    + bg_ref[...], 0.0)
    s = jnp.sum(h, axis=0, keepdims=True)

    @pl.when(j == 0)
    def _():
        acc_ref[0:1, :] = s

    @pl.when(j > 0)
    def _():
        acc_ref[0:1, :] = acc_ref[0:1, :] + s

    @pl.when(j == nblk - 1)
    def _():
        pooled = acc_ref[0:1, :] * (1.0 / N_NODES)
        z = jnp.dot(pooled, w1_ref[...],
                    preferred_element_type=jnp.float32) + b1_ref[...]
        z = jnp.dot(z, w2_ref[...],
                    preferred_element_type=jnp.float32) + b2_ref[...]
        m = jnp.max(z)
        lse = m + jnp.log(jnp.sum(jnp.exp(z - m)))
        o_ref[...] = z - lse


def _final(p0, p1, d0, d1, W_gcn, b_gcn, W_fuse1, b_fuse1, W_fuse2, b_fuse2):
    nblk = 10
    rows = N_NODES // nblk
    return pl.pallas_call(
        functools.partial(_final_body, nblk=nblk),
        grid=(nblk,),
        in_specs=[
            pl.BlockSpec((rows, D_FEAT), lambda j: (j, 0)),
            pl.BlockSpec((rows, D_FEAT), lambda j: (j, 0)),
            pl.BlockSpec((rows, 1), lambda j: (j, 0)),
            pl.BlockSpec((rows, 1), lambda j: (j, 0)),
            pl.BlockSpec((D_FEAT, D_FEAT), lambda j: (0, 0)),
            pl.BlockSpec((1, D_FEAT), lambda j: (0, 0)),
            pl.BlockSpec((D_FEAT, N_ANS), lambda j: (0, 0)),
            pl.BlockSpec((1, N_ANS), lambda j: (0, 0)),
            pl.BlockSpec((N_ANS, N_ANS), lambda j: (0, 0)),
            pl.BlockSpec((1, N_ANS), lambda j: (0, 0)),
        ],
        out_specs=pl.BlockSpec((1, N_ANS), lambda j: (0, 0)),
        out_shape=jax.ShapeDtypeStruct((1, N_ANS), jnp.float32),
        scratch_shapes=[pltpu.VMEM((8, D_FEAT), jnp.float32)],
    )(p0, p1, d0, d1, W_gcn, b_gcn, W_fuse1, b_fuse1, W_fuse2, b_fuse2)


def kernel(x, edge_index, W_gcn, b_gcn, W_fuse1, b_fuse1, W_fuse2, b_fuse2):
    src = edge_index[0].astype(jnp.int32)
    dst = edge_index[1].astype(jnp.int32)
    pad = TOT - N_EDGES
    srcp = jnp.concatenate([src, jnp.arange(pad, dtype=jnp.int32) % N_NODES])
    trash = N_NODES + jnp.arange(pad, dtype=jnp.int32) % (R - N_NODES)
    dstp = jnp.concatenate([dst, trash])
    src_a = srcp.reshape(TOT // BA, BA)
    dst_a = dstp.reshape(TOT // BA, BA)
    dst_b = dstp.reshape(BT, B)

    ones_deg = jnp.ones((B,), jnp.float32)
    zeros_deg = jnp.zeros((R,), jnp.float32)

    deg_kernel, agg_kernel = _sc_kernels()
    d0g, d1g = deg_kernel(dst_b, ones_deg, zeros_deg)
    d0 = d0g.reshape(R, 1)
    d1 = d1g.reshape(R, 1)

    xs = _scale_x(x, d0, d1)

    aggp = agg_kernel(src_a, dst_a, xs)

    return _final(aggp[0], aggp[1], d0, d1,
                  W_gcn, b_gcn.reshape(1, D_FEAT),
                  W_fuse1, b_fuse1.reshape(1, N_ANS),
                  W_fuse2, b_fuse2.reshape(1, N_ANS))
